# baseline (speedup 1.0000x reference)
import functools

import jax
import jax.numpy as jnp
from jax.experimental import pallas as pl
from jax.experimental.pallas import tpu as pltpu

B, L, D = 4, 8192, 768
TILE_L = 1024
NL = L // TILE_L
L4 = L // 4
L16 = L // 16
PT = L16
NP = L4 // PT
TEMP_INV = 2.0


def _fused_kernel(x_ref, g_ref, w1_ref, b1_ref, w2_ref, b2_ref, p4_ref,
                  wp4_ref, bp4_ref, wp16_ref, bp16_ref, p16_ref,
                  out_ref, xm4_s, s0_s, s1_s):
    t = pl.program_id(1)

    @pl.when(t < NL)
    def _router_step():
        x = jnp.clip(x_ref[0], -4.0, 4.0)
        xb = x.astype(jnp.bfloat16)
        pooled = jnp.dot(p4_ref[...], xb,
                         preferred_element_type=jnp.float32).astype(jnp.bfloat16)
        xm4_s[pl.ds(t * (TILE_L // 4), TILE_L // 4), :] = pooled
        h = jnp.dot(xb, w1_ref[...], preferred_element_type=jnp.float32) + b1_ref[0]
        h = jnp.maximum(h, 0.0).astype(jnp.bfloat16)
        logits = jnp.dot(h, w2_ref[...], preferred_element_type=jnp.float32) + b2_ref[0]
        lc = jnp.clip(logits, -15.0, 15.0)
        lt = lc.T
        z = (lt + g_ref[0]) * TEMP_INV
        d = z[0:1, :] - z[1:2, :]
        p0 = 1.0 / (1.0 + jnp.exp(-d))
        p0c = jnp.clip(p0, 1e-7, 1.0 - 1e-7)
        p1c = jnp.clip(1.0 - p0, 1e-7, 1.0 - 1e-7)
        s0 = p0c[:, 0:128]
        s1 = p1c[:, 0:128]
        for k in range(1, TILE_L // 128):
            s0 = s0 + p0c[:, k * 128:(k + 1) * 128]
            s1 = s1 + p1c[:, k * 128:(k + 1) * 128]
        s0_s[0, :] = jnp.where(t == 0, s0[0], s0_s[0, :] + s0[0])
        s1_s[0, :] = jnp.where(t == 0, s1[0], s1_s[0, :] + s1[0])

    @pl.when(t >= NL)
    def _proj_step():
        pt = t - NL
        m4 = jnp.sum(s0_s[0, :])
        m16 = jnp.sum(s1_s[0, :])
        denom = m4 + m16 + 1e-7 * L
        w4 = m4 / denom
        w16 = m16 / denom
        rows = xm4_s[pl.ds(pt * PT, PT), :]
        y4 = jnp.dot(rows, wp4_ref[...],
                     preferred_element_type=jnp.float32) + bp4_ref[0]
        y4 = jnp.clip(y4, -6.0, 6.0)

        @pl.when(pt == 0)
        def _mixed_tile():
            xm16 = jnp.dot(p16_ref[...], xm4_s[...],
                           preferred_element_type=jnp.float32).astype(jnp.bfloat16)
            y16 = jnp.dot(xm16, wp16_ref[...],
                          preferred_element_type=jnp.float32) + bp16_ref[0]
            y16 = jnp.clip(y16, -6.0, 6.0)
            out_ref[0] = jnp.clip(w4 * y4 + w16 * y16, -6.0, 6.0)

        @pl.when(pt != 0)
        def _pure_tile():
            out_ref[0] = jnp.clip(w4 * y4, -6.0, 6.0)


@functools.partial(jax.jit, static_argnames=())
def kernel(visual_tokens, W1, b1, W2, b2, Wp4, bp4, Wp16, bp16):
    f32 = jnp.float32
    gkey = jax.random.key(42)
    u = jax.random.uniform(gkey, (B, L, 2), minval=1e-7, maxval=1.0 - 1e-7)
    gumbel = jnp.clip(-jnp.log(-jnp.log(u)), -6.0, 6.0)
    gt = jnp.transpose(gumbel, (0, 2, 1))

    w1t = W1.T.astype(jnp.bfloat16)
    w2t = W2.T.astype(jnp.bfloat16)
    wp4t = Wp4.T.astype(jnp.bfloat16)
    wp16t = Wp16.T.astype(jnp.bfloat16)
    b1r = b1.reshape(1, D).astype(f32)
    b2r = b2.reshape(1, 2).astype(f32)
    bp4r = bp4.reshape(1, D).astype(f32)
    bp16r = bp16.reshape(1, D).astype(f32)

    def _pool_matrix(rows, cols):
        sel = jnp.arange(rows)[:, None] == (jnp.arange(cols)[None, :] // 4)
        return jnp.where(sel, 0.25, 0.0).astype(jnp.bfloat16)

    p4 = _pool_matrix(TILE_L // 4, TILE_L)
    p16 = _pool_matrix(L16, L4)

    const = lambda b, t: (0, 0)
    out = pl.pallas_call(
        _fused_kernel,
        grid=(B, NL + NP),
        in_specs=[
            pl.BlockSpec((1, TILE_L, D),
                         lambda b, t: (b, jnp.minimum(t, NL - 1), 0)),
            pl.BlockSpec((1, 2, TILE_L),
                         lambda b, t: (b, 0, jnp.minimum(t, NL - 1))),
            pl.BlockSpec((D, D), const),
            pl.BlockSpec((1, D), const),
            pl.BlockSpec((D, 2), const),
            pl.BlockSpec((1, 2), const),
            pl.BlockSpec((TILE_L // 4, TILE_L), const),
            pl.BlockSpec((D, D), const),
            pl.BlockSpec((1, D), const),
            pl.BlockSpec((D, D), const),
            pl.BlockSpec((1, D), const),
            pl.BlockSpec((L16, L4), const),
        ],
        out_specs=pl.BlockSpec((1, PT, D),
                               lambda b, t: (b, jnp.maximum(t - NL, 0), 0)),
        out_shape=jax.ShapeDtypeStruct((B, L4, D), f32),
        scratch_shapes=[
            pltpu.VMEM((L4, D), jnp.bfloat16),
            pltpu.VMEM((1, 128), f32),
            pltpu.VMEM((1, 128), f32),
        ],
    )(visual_tokens, gt, w1t, b1r, w2t, b2r, p4, wp4t, bp4r, wp16t, bp16r, p16)
    return out
